# baseline (speedup 1.0000x reference)
import functools

import jax
import jax.numpy as jnp
from jax.experimental import pallas as pl
from jax.experimental.pallas import tpu as pltpu

N_CHANNELS = 4096
N_ENERGIES = 8192
BE = 256


def _matvec_kernel(params_ref, energies_ref, tm_ref, out_ref, acc_ref):
    j = pl.program_id(0)
    alpha = params_ref[0, 0]
    norm = params_ref[0, 1]
    oma = 1.0 - alpha
    e_lo = energies_ref[0, :]
    e_hi = energies_ref[1, :]
    flux = (norm / oma) * (
        jnp.exp(oma * jnp.log(e_hi)) - jnp.exp(oma * jnp.log(e_lo))
    )
    partial = jnp.dot(
        tm_ref[...], flux.reshape(BE, 1), preferred_element_type=jnp.float32
    )

    @pl.when(j == 0)
    def _init():
        acc_ref[...] = partial

    @pl.when(j > 0)
    def _acc():
        acc_ref[...] += partial

    @pl.when(j == pl.num_programs(0) - 1)
    def _fin():
        out_ref[...] = jnp.maximum(acc_ref[...], 1e-6)


def kernel(parameters, energies, transfer_matrix):
    params2d = parameters.reshape(1, 2)
    grid = N_ENERGIES // BE
    out = pl.pallas_call(
        _matvec_kernel,
        grid=(grid,),
        in_specs=[
            pl.BlockSpec((1, 2), lambda j: (0, 0), memory_space=pltpu.SMEM),
            pl.BlockSpec((2, BE), lambda j: (0, j)),
            pl.BlockSpec((N_CHANNELS, BE), lambda j: (0, j)),
        ],
        out_specs=pl.BlockSpec((N_CHANNELS, 1), lambda j: (0, 0)),
        out_shape=jax.ShapeDtypeStruct((N_CHANNELS, 1), jnp.float32),
        scratch_shapes=[pltpu.VMEM((N_CHANNELS, 1), jnp.float32)],
    )(params2d, energies, transfer_matrix)
    return out.reshape(N_CHANNELS)


# speedup vs baseline: 1.2457x; 1.2457x over previous
import functools

import jax
import jax.numpy as jnp
from jax.experimental import pallas as pl
from jax.experimental.pallas import tpu as pltpu

N_CHANNELS = 4096
N_ENERGIES = 8192
BC = 512


def _matvec_kernel(params_ref, energies_ref, tm_ref, out_ref):
    alpha = params_ref[0, 0]
    norm = params_ref[0, 1]
    oma = 1.0 - alpha
    e_lo = energies_ref[0, :]
    e_hi = energies_ref[1, :]
    flux = (norm / oma) * (
        jnp.exp(oma * jnp.log(e_hi)) - jnp.exp(oma * jnp.log(e_lo))
    )
    res = jnp.dot(
        tm_ref[...], flux.reshape(N_ENERGIES, 1), preferred_element_type=jnp.float32
    )
    out_ref[...] = jnp.maximum(res, 1e-6)


def kernel(parameters, energies, transfer_matrix):
    params2d = parameters.reshape(1, 2)
    grid = N_CHANNELS // BC
    out = pl.pallas_call(
        _matvec_kernel,
        grid=(grid,),
        in_specs=[
            pl.BlockSpec((1, 2), lambda i: (0, 0), memory_space=pltpu.SMEM),
            pl.BlockSpec((2, N_ENERGIES), lambda i: (0, 0)),
            pl.BlockSpec((BC, N_ENERGIES), lambda i: (i, 0)),
        ],
        out_specs=pl.BlockSpec((BC, 1), lambda i: (i, 0)),
        out_shape=jax.ShapeDtypeStruct((N_CHANNELS, 1), jnp.float32),
    )(params2d, energies, transfer_matrix)
    return out.reshape(N_CHANNELS)
